# baseline (speedup 1.0000x reference)
import jax
import jax.numpy as jnp
from jax.experimental import pallas as pl
from jax.experimental.pallas import tpu as pltpu

_BATCH = 16384
_IN_COLS = 40
_OUT_COLS = 69
_BLKC = 16384


def _tc_body(wcat_ref, in_ref, out_ref):
    x = in_ref[...]
    ids = jnp.round(x[0:3, :]).astype(jnp.int32)
    b0 = jnp.minimum(ids[0:1, :], 64)
    b1 = jnp.minimum(ids[1:2, :], 32) + 65
    b2 = jnp.minimum(ids[2:3, :], 16) + 98
    sub = jax.lax.broadcasted_iota(jnp.int32, (128, _BLKC), 0)
    oh = ((sub == b0) | (sub == b1) | (sub == b2)).astype(jnp.float32)
    emb = jnp.dot(wcat_ref[...], oh, preferred_element_type=jnp.float32)
    out_ref[0:37, :] = x[3:40, :]
    out_ref[37:69, :] = emb


def kernel(inputs, W0, W1, W2):
    wcat = jnp.zeros((32, 128), jnp.float32)
    wcat = wcat.at[0:16, 0:65].set(W0.T)
    wcat = wcat.at[16:24, 65:98].set(W1.T)
    wcat = wcat.at[24:32, 98:115].set(W2.T)
    outT = pl.pallas_call(
        _tc_body,
        grid=(_BATCH // _BLKC,),
        in_specs=[
            pl.BlockSpec((32, 128), lambda i: (0, 0)),
            pl.BlockSpec((_IN_COLS, _BLKC), lambda i: (0, i)),
        ],
        out_specs=pl.BlockSpec((_OUT_COLS, _BLKC), lambda i: (0, i)),
        out_shape=jax.ShapeDtypeStruct((_OUT_COLS, _BATCH), jnp.float32),
        compiler_params=pltpu.CompilerParams(
            dimension_semantics=("arbitrary",)),
    )(wcat, inputs.T)
    return outT.T
